# baseline (speedup 1.0000x reference)
import jax
import jax.numpy as jnp
from jax import lax
from jax.experimental import pallas as pl
from jax.experimental.pallas import tpu as pltpu
from jax.experimental.pallas import tpu_sc as plsc

_NUM_CORES = 2
_NUM_SUBCORES = 16
_NUM_WORKERS = _NUM_CORES * _NUM_SUBCORES
_LANES = 16

_SEQ = 50
_BATCH = 16384
_DIM = 32
_BLK = 128
_BPR = 128 // _DIM
_NBLOCKS = _SEQ * (_BATCH // _BLK)
_BLOCKS_PER_W = _NBLOCKS // _NUM_WORKERS


def _body(table_hbm, tok_hbm, out_hbm, tv, gx, cv, gbuf, outv,
          sem_t, sem_g, sem_o):
  wid = lax.axis_index("s") * _NUM_CORES + lax.axis_index("c")
  base = wid * _BLOCKS_PER_W
  iota16 = lax.iota(jnp.int32, _LANES)

  def block_jbc(i):
    blk = jnp.minimum(base + i, _NBLOCKS - 1)
    return blk // (_BATCH // _BLK), blk % (_BATCH // _BLK)

  def tok_start(i, par):
    j, bc = block_jbc(i)
    pltpu.async_copy(tok_hbm.at[pl.ds(j * _BATCH + bc * _BLK, _BLK)],
                     tv.at[par], sem_t.at[par])

  def tok_wait(par):
    pltpu.make_async_copy(tok_hbm.at[pl.ds(0, _BLK)], tv.at[par],
                          sem_t.at[par]).wait()

  def gather_wait(par):
    pltpu.make_async_copy(table_hbm.at[gx.at[par]], gbuf.at[par],
                          sem_g.at[par]).wait()

  def out_wait(par):
    for tr in range(_DIM // 8):
      pltpu.make_async_copy(outv.at[par, pl.ds(tr * 8, 8)],
                            out_hbm.at[0, pl.ds(tr * 8, 8), pl.ds(0, _BLK)],
                            sem_o.at[par]).wait()

  def stage_a(i, par):
    tok_wait(par)
    for m in range(_BLK // _LANES):
      e = tv[par, pl.ds(m * _LANES, _LANES)]
      gx[par, pl.ds(m * _LANES, _LANES)] = lax.shift_right_logical(
          e, _BPR // 2)
      cv[par, pl.ds(m * _LANES, _LANES)] = lax.shift_left(
          lax.bitwise_and(e, _BPR - 1), 5)
    pltpu.async_copy(table_hbm.at[gx.at[par]], gbuf.at[par], sem_g.at[par])
    tok_start(i + 2, par)

  def stage_b(i, par, first):
    gather_wait(par)
    if not first:
      out_wait(par)
    for m in range(_BLK // _LANES):
      idx_l = iota16 + (m * _LANES)
      c_m = cv[par, pl.ds(m * _LANES, _LANES)]
      for rr in range(_DIM):
        outv[par, rr, pl.ds(m * _LANES, _LANES)] = plsc.load_gather(
            gbuf.at[par], [idx_l, c_m + rr])
    j, bc = block_jbc(i)
    for tr in range(_DIM // 8):
      pltpu.async_copy(outv.at[par, pl.ds(tr * 8, 8)],
                       out_hbm.at[j, pl.ds(tr * 8, 8), pl.ds(bc * _BLK, _BLK)],
                       sem_o.at[par])

  tok_start(0, 0)
  tok_start(1, 1)
  stage_a(0, 0)
  stage_a(1, 1)
  stage_b(0, 0, first=True)
  stage_a(2, 0)
  stage_b(1, 1, first=True)
  stage_a(3, 1)

  def step(p, carry):
    i = 2 * p
    stage_b(i, 0, first=False)
    stage_a(i + 2, 0)
    stage_b(i + 1, 1, first=False)
    stage_a(i + 3, 1)
    return carry

  lax.fori_loop(1, _BLOCKS_PER_W // 2 - 1, step, 0, unroll=False)

  i = _BLOCKS_PER_W - 2
  stage_b(i, 0, first=False)
  stage_b(i + 1, 1, first=False)
  for par in (0, 1):
    out_wait(par)
    tok_wait(par)


def _make_kernel():
  mesh = plsc.VectorSubcoreMesh(core_axis_name="c", subcore_axis_name="s")
  return pl.kernel(
      _body,
      out_type=jax.ShapeDtypeStruct((_SEQ, _DIM, _BATCH), jnp.float32),
      mesh=mesh,
      scratch_types=[
          pltpu.VMEM((2, _BLK), jnp.int32),
          pltpu.VMEM((2, _BLK), jnp.int32),
          pltpu.VMEM((2, _BLK), jnp.int32),
          pltpu.VMEM((2, _BLK, 128), jnp.float32),
          pltpu.VMEM((2, _DIM, _BLK), jnp.float32),
          pltpu.SemaphoreType.DMA((2,)),
          pltpu.SemaphoreType.DMA((2,)),
          pltpu.SemaphoreType.DMA((2,)),
      ],
      compiler_params=pltpu.CompilerParams(use_tc_tiling_on_sc=True,
                                           needs_layout_passes=False),
  )


@jax.jit
def kernel(token_ids, weight):
  b, s = token_ids.shape
  dim = weight.shape[1]
  table = (weight * jnp.float32(1.0)).reshape(weight.shape[0] * dim // 128, 128)
  tok = token_ids.T.reshape(b * s).astype(jnp.int32)
  outt = _make_kernel()(table, tok)
  return outt.transpose(2, 0, 1)


# speedup vs baseline: 1.1302x; 1.1302x over previous
import jax
import jax.numpy as jnp
from jax import lax
from jax.experimental import pallas as pl
from jax.experimental.pallas import tpu as pltpu
from jax.experimental.pallas import tpu_sc as plsc

_NUM_CORES = 2
_NUM_SUBCORES = 16
_NUM_WORKERS = _NUM_CORES * _NUM_SUBCORES
_LANES = 16

_SEQ = 50
_BATCH = 16384
_DIM = 32
_BLK = 128
_BPR = 128 // _DIM
_NBLOCKS = _SEQ * (_BATCH // _BLK)
_BLOCKS_PER_W = _NBLOCKS // _NUM_WORKERS


def _body(table_hbm, tok_hbm, out_hbm, tv, gx, cv, gbuf, outv,
          sem_t, sem_g, sem_o):
  wid = lax.axis_index("s") * _NUM_CORES + lax.axis_index("c")
  base = wid * _BLOCKS_PER_W
  iota16 = lax.iota(jnp.int32, _LANES)

  def block_jbc(i):
    blk = jnp.minimum(base + i, _NBLOCKS - 1)
    return blk // (_BATCH // _BLK), blk % (_BATCH // _BLK)

  def tok_start(i, par):
    j, bc = block_jbc(i)
    pltpu.async_copy(tok_hbm.at[pl.ds(j * _BATCH + bc * _BLK, _BLK)],
                     tv.at[par], sem_t.at[par])

  def tok_wait(par):
    pltpu.make_async_copy(tok_hbm.at[pl.ds(0, _BLK)], tv.at[par],
                          sem_t.at[par]).wait()

  def gather_wait(par):
    pltpu.make_async_copy(table_hbm.at[gx.at[par]], gbuf.at[par],
                          sem_g.at[par]).wait()

  def out_wait(par):
    for tr in range(_DIM // 8):
      pltpu.make_async_copy(outv.at[par, pl.ds(tr * 8, 8)],
                            out_hbm.at[0, pl.ds(tr * 8, 8), pl.ds(0, _BLK)],
                            sem_o.at[par]).wait()

  def stage_a(i, par):
    tok_wait(par)
    for m in range(_BLK // _LANES):
      e = tv[par, pl.ds(m * _LANES, _LANES)]
      gx[par, pl.ds(m * _LANES, _LANES)] = lax.shift_right_logical(
          e, _BPR // 2)
      cv[par, pl.ds(m * _LANES, _LANES)] = lax.shift_left(
          lax.bitwise_and(e, _BPR - 1), 5)
    pltpu.async_copy(table_hbm.at[gx.at[par]], gbuf.at[par], sem_g.at[par])
    tok_start(i + 2, par)

  def stage_b(i, par, first):
    gather_wait(par)
    if not first:
      out_wait(par)
    for m in range(_BLK // _LANES):
      c_m = cv[par, pl.ds(m * _LANES, _LANES)]
      for t2 in range(_LANES):
        t = m * _LANES + t2
        c_sc = c_m[t2]
        for k in range(_DIM // _LANES):
          v = gbuf[par, t, pl.ds(c_sc + k * _LANES, _LANES)]
          plsc.store_scatter(
              outv.at[par],
              [iota16 + k * _LANES, jnp.full((_LANES,), t, jnp.int32)], v)
    j, bc = block_jbc(i)
    for tr in range(_DIM // 8):
      pltpu.async_copy(outv.at[par, pl.ds(tr * 8, 8)],
                       out_hbm.at[j, pl.ds(tr * 8, 8), pl.ds(bc * _BLK, _BLK)],
                       sem_o.at[par])

  tok_start(0, 0)
  tok_start(1, 1)
  stage_a(0, 0)
  stage_a(1, 1)
  stage_b(0, 0, first=True)
  stage_a(2, 0)
  stage_b(1, 1, first=True)
  stage_a(3, 1)

  def step(p, carry):
    i = 2 * p
    stage_b(i, 0, first=False)
    stage_a(i + 2, 0)
    stage_b(i + 1, 1, first=False)
    stage_a(i + 3, 1)
    return carry

  lax.fori_loop(1, _BLOCKS_PER_W // 2 - 1, step, 0, unroll=False)

  i = _BLOCKS_PER_W - 2
  stage_b(i, 0, first=False)
  stage_b(i + 1, 1, first=False)
  for par in (0, 1):
    out_wait(par)
    tok_wait(par)


def _make_kernel():
  mesh = plsc.VectorSubcoreMesh(core_axis_name="c", subcore_axis_name="s")
  return pl.kernel(
      _body,
      out_type=jax.ShapeDtypeStruct((_SEQ, _DIM, _BATCH), jnp.float32),
      mesh=mesh,
      scratch_types=[
          pltpu.VMEM((2, _BLK), jnp.int32),
          pltpu.VMEM((2, _BLK), jnp.int32),
          pltpu.VMEM((2, _BLK), jnp.int32),
          pltpu.VMEM((2, _BLK, 128), jnp.float32),
          pltpu.VMEM((2, _DIM, _BLK), jnp.float32),
          pltpu.SemaphoreType.DMA((2,)),
          pltpu.SemaphoreType.DMA((2,)),
          pltpu.SemaphoreType.DMA((2,)),
      ],
      compiler_params=pltpu.CompilerParams(use_tc_tiling_on_sc=True,
                                           needs_layout_passes=False),
  )


@jax.jit
def kernel(token_ids, weight):
  b, s = token_ids.shape
  dim = weight.shape[1]
  table = (weight * jnp.float32(1.0)).reshape(weight.shape[0] * dim // 128, 128)
  tok = token_ids.T.reshape(b * s).astype(jnp.int32)
  outt = _make_kernel()(table, tok)
  return outt.transpose(2, 0, 1)


# speedup vs baseline: 1.2719x; 1.1253x over previous
import jax
import jax.numpy as jnp
from jax import lax
from jax.experimental import pallas as pl
from jax.experimental.pallas import tpu as pltpu
from jax.experimental.pallas import tpu_sc as plsc

_NUM_CORES = 2
_NUM_SUBCORES = 16
_NUM_WORKERS = _NUM_CORES * _NUM_SUBCORES
_LANES = 16

_SEQ = 50
_BATCH = 16384
_DIM = 32
_BLK = 128
_BPR = 128 // _DIM
_NBLOCKS = _SEQ * (_BATCH // _BLK)
_BLOCKS_PER_W = _NBLOCKS // _NUM_WORKERS


def _body(table_hbm, tok_hbm, out_hbm, tv, gx, cv, gbuf, outv,
          sem_t, sem_g, sem_o):
  wid = lax.axis_index("s") * _NUM_CORES + lax.axis_index("c")
  base = wid * _BLOCKS_PER_W
  iota16 = lax.iota(jnp.int32, _LANES)

  def block_jbc(i):
    blk = jnp.minimum(base + i, _NBLOCKS - 1)
    return blk // (_BATCH // _BLK), blk % (_BATCH // _BLK)

  def tok_start(i, par):
    j, bc = block_jbc(i)
    pltpu.async_copy(tok_hbm.at[pl.ds(j * _BATCH + bc * _BLK, _BLK)],
                     tv.at[par], sem_t.at[par])

  def tok_wait(par):
    pltpu.make_async_copy(tok_hbm.at[pl.ds(0, _BLK)], tv.at[par],
                          sem_t.at[par]).wait()

  def gather_wait(par):
    pltpu.make_async_copy(table_hbm.at[gx.at[par]], gbuf.at[par],
                          sem_g.at[par]).wait()

  def out_wait(par):
    for tr in range(_DIM // 8):
      pltpu.make_async_copy(outv.at[par, pl.ds(tr * 8, 8)],
                            out_hbm.at[0, pl.ds(tr * 8, 8), pl.ds(0, _BLK)],
                            sem_o.at[par]).wait()

  def stage_a(i, par):
    tok_wait(par)
    for m in range(_BLK // _LANES):
      e = tv[par, pl.ds(m * _LANES, _LANES)]
      gx[par, pl.ds(m * _LANES, _LANES)] = lax.shift_right_logical(
          e, _BPR // 2)
      cv[par, pl.ds(m * _LANES, _LANES)] = lax.shift_left(
          lax.bitwise_and(e, _BPR - 1), 5)
    pltpu.async_copy(table_hbm.at[gx.at[par]], gbuf.at[par], sem_g.at[par])
    tok_start(i + 2, par)

  def stage_b(i, par, first):
    gather_wait(par)
    if not first:
      out_wait(par)
    for m in range(_BLK // _LANES):
      c_m = cv[par, pl.ds(m * _LANES, _LANES)]
      for t2 in range(0, _LANES, 2):
        ta = m * _LANES + t2
        tb = ta + 1
        ca = c_m[t2]
        cb = c_m[t2 + 1]
        vs = []
        for k in range(_DIM // _LANES):
          vs.append((k, ta, gbuf[par, ta, pl.ds(ca + k * _LANES, _LANES)]))
          vs.append((k, tb, gbuf[par, tb, pl.ds(cb + k * _LANES, _LANES)]))
        for k, t, v in vs:
          plsc.store_scatter(
              outv.at[par],
              [iota16 + k * _LANES, jnp.full((_LANES,), t, jnp.int32)], v)
    j, bc = block_jbc(i)
    for tr in range(_DIM // 8):
      pltpu.async_copy(outv.at[par, pl.ds(tr * 8, 8)],
                       out_hbm.at[j, pl.ds(tr * 8, 8), pl.ds(bc * _BLK, _BLK)],
                       sem_o.at[par])

  tok_start(0, 0)
  tok_start(1, 1)
  stage_a(0, 0)
  stage_a(1, 1)
  stage_b(0, 0, first=True)
  stage_a(2, 0)
  stage_b(1, 1, first=True)
  stage_a(3, 1)

  def step(p, carry):
    i = 2 * p
    stage_b(i, 0, first=False)
    stage_a(i + 2, 0)
    stage_b(i + 1, 1, first=False)
    stage_a(i + 3, 1)
    return carry

  lax.fori_loop(1, _BLOCKS_PER_W // 2 - 1, step, 0, unroll=False)

  i = _BLOCKS_PER_W - 2
  stage_b(i, 0, first=False)
  stage_b(i + 1, 1, first=False)
  for par in (0, 1):
    out_wait(par)
    tok_wait(par)


def _make_kernel():
  mesh = plsc.VectorSubcoreMesh(core_axis_name="c", subcore_axis_name="s")
  return pl.kernel(
      _body,
      out_type=jax.ShapeDtypeStruct((_SEQ, _DIM, _BATCH), jnp.float32),
      mesh=mesh,
      scratch_types=[
          pltpu.VMEM((2, _BLK), jnp.int32),
          pltpu.VMEM((2, _BLK), jnp.int32),
          pltpu.VMEM((2, _BLK), jnp.int32),
          pltpu.VMEM((2, _BLK, 128), jnp.float32),
          pltpu.VMEM((2, _DIM, _BLK), jnp.float32),
          pltpu.SemaphoreType.DMA((2,)),
          pltpu.SemaphoreType.DMA((2,)),
          pltpu.SemaphoreType.DMA((2,)),
      ],
      compiler_params=pltpu.CompilerParams(use_tc_tiling_on_sc=True,
                                           needs_layout_passes=False),
  )


@jax.jit
def kernel(token_ids, weight):
  b, s = token_ids.shape
  dim = weight.shape[1]
  table = (weight * jnp.float32(1.0)).reshape(weight.shape[0] * dim // 128, 128)
  tok = token_ids.T.reshape(b * s).astype(jnp.int32)
  outt = _make_kernel()(table, tok)
  return outt.transpose(2, 0, 1)
